# baseline (speedup 1.0000x reference)
import jax
import jax.numpy as jnp
from jax import lax
from jax.experimental import pallas as pl
from jax.experimental.pallas import tpu as pltpu
from jax.experimental.pallas import tpu_sc as plsc

RES = 128
THRESH = 0.01
N = 2097152
N_CELLS = RES * RES * RES
SENTINEL = 2 * N_CELLS
QCELLS = N_CELLS // 4
SPAD = 8192

NUM_TILES = 16
NPT = N // NUM_TILES
CHUNK = 8192
NCHUNK = NPT // CHUNK
QSLAB = QCELLS // NUM_TILES

PB = 1024


def _idx_body(coords_ref, dens_ref, idx_ref):
    x = coords_ref[:, 0]
    y = coords_ref[:, 1]
    z = coords_ref[:, 2]
    ix = jnp.clip((x * 127.0).astype(jnp.int32), 0, RES - 1)
    iy = jnp.clip((y * 127.0).astype(jnp.int32), 0, RES - 1)
    iz = jnp.clip((z * 127.0).astype(jnp.int32), 0, RES - 1)
    lin = (ix * RES + iy) * RES + iz
    occ = dens_ref[...] > THRESH
    idx_ref[...] = jnp.where(occ, lin, SENTINEL)


_index_kernel = pl.pallas_call(
    _idx_body,
    out_shape=jax.ShapeDtypeStruct((N,), jnp.int32),
    grid=(N // PB,),
    in_specs=[
        pl.BlockSpec((PB, 3), lambda i: (i, 0)),
        pl.BlockSpec((PB,), lambda i: (i,)),
    ],
    out_specs=pl.BlockSpec((PB,), lambda i: (i,)),
)


def _body(idx_ref, grid_ref, out_ref,
          iv, idxbuf0, idxbuf1, ones, gshared, sem_in, sem_scat):
    cid = lax.axis_index("c")
    sid = lax.axis_index("s")

    @pl.loop(0, CHUNK // 16)
    def _fill(r):
        ones[pl.ds(r * 16, 16)] = jnp.full((16,), 1.0, jnp.float32)

    def start_in(c, b):
        base = sid * NPT + c * CHUNK
        pltpu.async_copy(idx_ref.at[pl.ds(base, CHUNK)],
                         iv.at[b], sem_in.at[b])

    def wait_in(b):
        pltpu.make_async_copy(idx_ref.at[pl.ds(0, CHUNK)],
                              iv.at[b], sem_in.at[b]).wait()

    def compute(b, q_base):
        idxbuf = idxbuf0 if b == 0 else idxbuf1

        @pl.loop(0, CHUNK // 128)
        def _row(r):
            for g in range(8):
                off = r * 128 + g * 16
                lin = iv[b, pl.ds(off, 16)]
                local = lin - q_base
                keep = (local >= 0) & (local < QCELLS)
                trash = QCELLS + (off & (SPAD - 1))
                idxbuf[pl.ds(off, 16)] = jnp.where(keep, local, trash)

    def fire_scat(b):
        idxbuf = idxbuf0 if b == 0 else idxbuf1
        pltpu.async_copy(ones, gshared.at[idxbuf], sem_scat.at[b])

    def drain_scat(b):
        idxbuf = idxbuf0 if b == 0 else idxbuf1
        pltpu.make_async_copy(ones, gshared.at[idxbuf],
                              sem_scat.at[b]).wait()

    NJ = NCHUNK // 2

    for p in range(2):
        q_base = (cid * 2 + p) * QCELLS

        pltpu.sync_copy(grid_ref.at[pl.ds(q_base + sid * QSLAB, QSLAB)],
                        gshared.at[pl.ds(sid * QSLAB, QSLAB)])
        plsc.subcore_barrier()

        start_in(0, 0)
        start_in(1, 1)

        @pl.loop(0, NJ)
        def _j(j):
            for h in range(2):
                c = 2 * j + h
                wait_in(h)

                @pl.when(j > 0)
                def _():
                    drain_scat(h)

                compute(h, q_base)
                fire_scat(h)

                @pl.when(j + 1 < NJ)
                def _():
                    start_in(c + 2, h)

        drain_scat(0)
        drain_scat(1)
        plsc.subcore_barrier()

        pltpu.sync_copy(gshared.at[pl.ds(sid * QSLAB, QSLAB)],
                        out_ref.at[pl.ds(q_base + sid * QSLAB, QSLAB)])
        if p == 0:
            plsc.subcore_barrier()


_mesh = plsc.VectorSubcoreMesh(
    core_axis_name="c", subcore_axis_name="s", num_cores=2)

_scatter = pl.kernel(
    _body,
    out_type=jax.ShapeDtypeStruct((N_CELLS,), jnp.float32),
    mesh=_mesh,
    scratch_types=[
        pltpu.VMEM((2, CHUNK), jnp.int32),
        pltpu.VMEM((CHUNK,), jnp.int32),
        pltpu.VMEM((CHUNK,), jnp.int32),
        pltpu.VMEM((CHUNK,), jnp.float32),
        pltpu.VMEM_SHARED((QCELLS + SPAD,), jnp.float32),
        pltpu.SemaphoreType.DMA((2,)),
        pltpu.SemaphoreType.DMA((2,)),
    ],
)


@jax.jit
def kernel(coords, densities, grid):
    idx = _index_kernel(coords, densities)
    out = _scatter(idx, grid.reshape(-1))
    return out.reshape(RES, RES, RES)


# speedup vs baseline: 10.3041x; 10.3041x over previous
import jax
import jax.numpy as jnp
from jax import lax
from jax.experimental import pallas as pl
from jax.experimental.pallas import tpu as pltpu
from jax.experimental.pallas import tpu_sc as plsc

RES = 128
THRESH = 0.01
N = 2097152
N_CELLS = RES * RES * RES
SENTINEL = 2 * N_CELLS
QCELLS = N_CELLS // 4
SPAD = 8192

NUM_TILES = 16
NPT = N // NUM_TILES
CHUNK = 8192
NCHUNK = NPT // CHUNK
QSLAB = QCELLS // NUM_TILES

PB = 1024


def _idx_body(coords_ref, dens_ref, idx_ref):
    x = coords_ref[:, 0]
    y = coords_ref[:, 1]
    z = coords_ref[:, 2]
    ix = jnp.clip((x * 127.0).astype(jnp.int32), 0, RES - 1)
    iy = jnp.clip((y * 127.0).astype(jnp.int32), 0, RES - 1)
    iz = jnp.clip((z * 127.0).astype(jnp.int32), 0, RES - 1)
    lin = (ix * RES + iy) * RES + iz
    occ = dens_ref[...] > THRESH
    idx_ref[...] = jnp.where(occ, lin, SENTINEL)


_index_kernel = pl.pallas_call(
    _idx_body,
    out_shape=jax.ShapeDtypeStruct((N,), jnp.int32),
    grid=(N // PB,),
    in_specs=[
        pl.BlockSpec((PB, 3), lambda i: (i, 0)),
        pl.BlockSpec((PB,), lambda i: (i,)),
    ],
    out_specs=pl.BlockSpec((PB,), lambda i: (i,)),
)


def _body(idx_ref, grid_ref, out_ref,
          iv, idxbuf0, idxbuf1, ones, gshared, sem_in, sem_scat):
    cid = lax.axis_index("c")
    sid = lax.axis_index("s")

    @pl.loop(0, CHUNK // 16)
    def _fill(r):
        ones[pl.ds(r * 16, 16)] = jnp.full((16,), 1.0, jnp.float32)

    def start_in(c, b):
        base = sid * NPT + c * CHUNK
        pltpu.async_copy(idx_ref.at[pl.ds(base, CHUNK)],
                         iv.at[b], sem_in.at[b])

    def wait_in(b):
        pltpu.make_async_copy(idx_ref.at[pl.ds(0, CHUNK)],
                              iv.at[b], sem_in.at[b]).wait()

    def compute(b, q_base):
        idxbuf = idxbuf0 if b == 0 else idxbuf1

        @pl.loop(0, CHUNK // 128)
        def _row(r):
            for g in range(8):
                off = r * 128 + g * 16
                lin = iv[b, pl.ds(off, 16)]
                local = lin - q_base
                keep = (local >= 0) & (local < QCELLS)
                trash = QCELLS + (off & (SPAD - 1))
                idxbuf[pl.ds(off, 16)] = jnp.where(keep, local, trash)

    def fire_scat(b):
        idxbuf = idxbuf0 if b == 0 else idxbuf1
        pltpu.async_copy(ones, gshared.at[idxbuf], sem_scat.at[b])

    def drain_scat(b):
        idxbuf = idxbuf0 if b == 0 else idxbuf1
        pltpu.make_async_copy(ones, gshared.at[idxbuf],
                              sem_scat.at[b]).wait()

    NJ = NCHUNK // 2

    for p in range(2):
        q_base = (cid * 2 + p) * QCELLS

        pltpu.sync_copy(grid_ref.at[pl.ds(q_base + sid * QSLAB, QSLAB)],
                        gshared.at[pl.ds(sid * QSLAB, QSLAB)])
        plsc.subcore_barrier()

        start_in(0, 0)
        start_in(1, 1)

        @pl.loop(0, NJ)
        def _j(j):
            for h in range(2):
                c = 2 * j + h
                wait_in(h)

                @pl.when(j > 0)
                def _():
                    drain_scat(h)

                compute(h, q_base)
                fire_scat(h)

                @pl.when(j + 1 < NJ)
                def _():
                    start_in(c + 2, h)

        drain_scat(0)
        drain_scat(1)
        plsc.subcore_barrier()

        pltpu.sync_copy(gshared.at[pl.ds(sid * QSLAB, QSLAB)],
                        out_ref.at[pl.ds(q_base + sid * QSLAB, QSLAB)])
        if p == 0:
            plsc.subcore_barrier()


_mesh = plsc.VectorSubcoreMesh(
    core_axis_name="c", subcore_axis_name="s", num_cores=2)

_scatter = pl.kernel(
    _body,
    out_type=jax.ShapeDtypeStruct((N_CELLS,), jnp.float32),
    mesh=_mesh,
    scratch_types=[
        pltpu.VMEM((2, CHUNK), jnp.int32),
        pltpu.VMEM((CHUNK,), jnp.int32),
        pltpu.VMEM((CHUNK,), jnp.int32),
        pltpu.VMEM((CHUNK,), jnp.float32),
        pltpu.VMEM_SHARED((QCELLS + SPAD,), jnp.float32),
        pltpu.SemaphoreType.DMA((2,)),
        pltpu.SemaphoreType.DMA((2,)),
    ],
)


@jax.jit
def kernel(coords, densities, grid):
    q = jnp.clip((coords * 127.0).astype(jnp.int32), 0, RES - 1)
    lin = (q[:, 0] * RES + q[:, 1]) * RES + q[:, 2]
    idx = jnp.where(densities > THRESH, lin, SENTINEL)
    out = _scatter(idx, grid.reshape(-1))
    return out.reshape(RES, RES, RES)
